# Initial kernel scaffold; baseline (speedup 1.0000x reference)
import jax
import jax.numpy as jnp
from jax.experimental import pallas as pl


def kernel(u, i, j, c, row, col, val, embeddings, W1_w, W1_b, W2_w, W2_b):
    raise NotImplementedError("write your pallas kernel here")


# speedup vs baseline: 10.2624x; 10.2624x over previous
import functools

import jax
import jax.numpy as jnp
from jax import lax
from jax.experimental import pallas as pl
from jax.experimental.pallas import tpu as pltpu
from jax.experimental.pallas import tpu_sc as plsc

N = 10000
E = 320000
D = 128
B = 16384

NC = 2
NS = 16
NW = NC * NS

EPT = E // NW
ECH = 128
NFULL = EPT // ECH
ETAIL = EPT - NFULL * ECH
ROWS_PER_SUB = 624
ROWS_TAIL = N - NS * ROWS_PER_SUB

BPT = B // NW
BCH = 64
NBCH = BPT // BCH

_mesh = plsc.VectorSubcoreMesh(core_axis_name="c", subcore_axis_name="s")


def _zeros16():
  return jnp.zeros((16,), jnp.float32)


@functools.partial(
    pl.kernel,
    out_type=jax.ShapeDtypeStruct((NC, N, D), jnp.float32),
    mesh=_mesh,
    compiler_params=pltpu.CompilerParams(needs_layout_passes=False),
    scratch_types=[
        [pltpu.VMEM((ECH, D), jnp.float32) for _ in range(3)],
        [pltpu.VMEM((ECH,), jnp.int32) for _ in range(3)],
        [pltpu.VMEM((ECH,), jnp.int32) for _ in range(3)],
        [pltpu.VMEM((ECH,), jnp.float32) for _ in range(3)],
        pltpu.VMEM((ETAIL,), jnp.int32),
        pltpu.VMEM((ETAIL,), jnp.int32),
        pltpu.VMEM((ETAIL,), jnp.float32),
        pltpu.VMEM_SHARED((N, D), jnp.float32),
        [pltpu.SemaphoreType.DMA for _ in range(3)],
        [pltpu.SemaphoreType.DMA for _ in range(3)],
        [pltpu.SemaphoreType.DMA for _ in range(3)],
        [pltpu.SemaphoreType.DMA for _ in range(3)],
        pltpu.SemaphoreType.DMA,
        pltpu.SemaphoreType.DMA,
        pltpu.SemaphoreType.DMA,
        pltpu.SemaphoreType.DMA,
    ],
)
def _sc_scatter(emb_hbm, col_hbm, row_hbm, val_hbm, out_hbm,
                rows, cidx, ridx, valc, cidxt, ridxt, valct, acc,
                gsem, ssem, csem, rsem, gsemt, ssemt, csemt, rsemt):
  cid = lax.axis_index("c")
  sid = lax.axis_index("s")
  wid = sid * NC + cid
  rows0 = rows[0]

  def _zbody(e, _):
    for k in range(D // 16):
      rows0[e, pl.ds(k * 16, 16)] = _zeros16()
    return _
  lax.fori_loop(0, ECH, _zbody, 0)

  arow = sid * ROWS_PER_SUB
  for m in range(ROWS_PER_SUB // ECH):
    pltpu.sync_copy(rows0, acc.at[pl.ds(arow + m * ECH, ECH)])
  rem = ROWS_PER_SUB - (ROWS_PER_SUB // ECH) * ECH
  if rem:
    pltpu.sync_copy(rows0.at[pl.ds(0, rem)],
                    acc.at[pl.ds(arow + (ROWS_PER_SUB // ECH) * ECH, rem)])

  @pl.when(sid == NS - 1)
  def _ztail():
    pltpu.sync_copy(rows0.at[pl.ds(0, ROWS_TAIL)],
                    acc.at[pl.ds(NS * ROWS_PER_SUB, ROWS_TAIL)])

  ebase = wid * EPT
  plsc.subcore_barrier()

  def _start_cv(ch, b):
    off = ebase + ch * ECH
    pltpu.async_copy(col_hbm.at[pl.ds(off, ECH)], cidx[b], csem[b])
    pltpu.async_copy(val_hbm.at[pl.ds(off, ECH)], valc[b], csem[b])

  def _wait_cv(ch, b):
    off = ebase + ch * ECH
    pltpu.make_async_copy(col_hbm.at[pl.ds(off, ECH)], cidx[b],
                          csem[b]).wait()
    pltpu.make_async_copy(val_hbm.at[pl.ds(off, ECH)], valc[b],
                          csem[b]).wait()

  def _start_ridx(ch, b):
    pltpu.async_copy(row_hbm.at[pl.ds(ebase + ch * ECH, ECH)], ridx[b],
                     rsem[b])

  def _wait_ridx(ch, b):
    pltpu.make_async_copy(row_hbm.at[pl.ds(ebase + ch * ECH, ECH)], ridx[b],
                          rsem[b]).wait()

  def _start_gather(b):
    pltpu.async_copy(emb_hbm.at[cidx[b]], rows[b], gsem[b])

  def _wait_gather(b):
    pltpu.make_async_copy(emb_hbm.at[cidx[b]], rows[b], gsem[b]).wait()

  def _start_scatter(b):
    pltpu.async_copy(rows[b], acc.at[ridx[b]], ssem[b], add=True)

  def _wait_scatter(b):
    pltpu.make_async_copy(rows[b], acc.at[ridx[b]], ssem[b]).wait()

  def _scale(b):
    rb = rows[b]
    vc = valc[b]

    @plsc.parallel_loop(0, ECH, unroll=8)
    def _body(e):
      vb = plsc.load_gather(vc, [lax.broadcast(e, (16,))])
      for k in range(D // 16):
        sl = (e, pl.ds(k * 16, 16))
        rb[sl] = rb[sl] * vb

  _start_cv(0, 0)
  _start_cv(1, 1)
  _wait_cv(0, 0)
  _start_ridx(0, 0)
  _start_gather(0)
  _wait_cv(1, 1)
  _start_ridx(1, 1)
  _start_gather(1)
  _start_cv(2, 2)
  _wait_gather(0)
  _scale(0)
  _wait_ridx(0, 0)
  _start_scatter(0)
  _wait_cv(2, 2)
  _start_ridx(2, 2)
  _start_gather(2)
  _start_cv(3, 0)
  _wait_gather(1)
  _scale(1)
  _wait_ridx(1, 1)
  _start_scatter(1)

  NITER = (NFULL - 3) // 3

  def _step(n, cur, nx1, nx2, k, tail_k):
    _wait_cv(n + 1, nx1)
    _wait_scatter(nx1)
    _start_ridx(n + 1, nx1)
    _start_gather(nx1)
    if tail_k is None:
      _start_cv(n + 2, nx2)
    else:
      @pl.when(k < tail_k)
      def _reg():
        _start_cv(n + 2, nx2)

      @pl.when(k == tail_k)
      def _tail():
        toff2 = ebase + NFULL * ECH
        pltpu.async_copy(col_hbm.at[pl.ds(toff2, ETAIL)], cidxt, csemt)
        pltpu.async_copy(val_hbm.at[pl.ds(toff2, ETAIL)], valct, csemt)
    _wait_gather(cur)
    _scale(cur)
    _wait_ridx(n, cur)
    _start_scatter(cur)

  def _iter(k, _):
    n0 = 3 * k + 2
    _step(n0, 2, 0, 1, k, None)
    _step(n0 + 1, 0, 1, 2, k, None)
    _step(n0 + 2, 1, 2, 0, k, NITER - 1)
    return _

  lax.fori_loop(0, NITER, _iter, 0)

  nl = NFULL - 1
  sl = nl % 3
  toff = ebase + NFULL * ECH
  pltpu.make_async_copy(col_hbm.at[pl.ds(toff, ETAIL)], cidxt, csemt).wait()
  pltpu.make_async_copy(val_hbm.at[pl.ds(toff, ETAIL)], valct, csemt).wait()
  _wait_scatter((nl + 1) % 3)
  rowst = rows[(nl + 1) % 3].at[pl.ds(0, ETAIL)]
  pltpu.async_copy(row_hbm.at[pl.ds(toff, ETAIL)], ridxt, rsemt)
  pltpu.async_copy(emb_hbm.at[cidxt], rowst, gsemt)
  _wait_gather(sl)
  _scale(sl)
  _wait_ridx(nl, sl)
  _start_scatter(sl)
  pltpu.make_async_copy(emb_hbm.at[cidxt], rowst, gsemt).wait()
  rbt = rows[(nl + 1) % 3]

  @plsc.parallel_loop(0, ETAIL, unroll=2)
  def _tscale(e):
    vb = plsc.load_gather(valct, [lax.broadcast(e, (16,))])
    for k in range(D // 16):
      sl2 = (e, pl.ds(k * 16, 16))
      rbt[sl2] = rbt[sl2] * vb

  pltpu.make_async_copy(row_hbm.at[pl.ds(toff, ETAIL)], ridxt, rsemt).wait()
  pltpu.async_copy(rowst, acc.at[ridxt], ssemt, add=True).wait()
  _wait_scatter((nl + 2) % 3)
  _wait_scatter(sl)

  plsc.subcore_barrier()

  pltpu.sync_copy(acc.at[pl.ds(arow, ROWS_PER_SUB)],
                  out_hbm.at[cid, pl.ds(arow, ROWS_PER_SUB)])

  @pl.when(sid == NS - 1)
  def _otail():
    pltpu.sync_copy(acc.at[pl.ds(NS * ROWS_PER_SUB, ROWS_TAIL)],
                    out_hbm.at[cid, pl.ds(NS * ROWS_PER_SUB, ROWS_TAIL)])


_BLK = 1000


def _tc_dense_body(p_ref, emb_ref, w1_ref, b1_ref, w2_ref, b2_ref,
                   act_ref, nrm_ref):
  t1e = p_ref[0] + p_ref[1]
  emb = emb_ref[...]
  t1 = jnp.dot(t1e, w1_ref[...], preferred_element_type=jnp.float32)
  t2 = jnp.dot(emb * t1e, w2_ref[...], preferred_element_type=jnp.float32)
  a = t1 + t2 + (b1_ref[...] + b2_ref[...])[None, :]
  act = jnp.where(a >= 0, a, 0.01 * a)
  act_ref[...] = act
  nr = jnp.sqrt(jnp.sum(act * act, axis=1, keepdims=True))
  nrm_ref[...] = act / jnp.maximum(nr, 1e-12)


def _tc_dense(parts, emb, w1t, b1, w2t, b2):
  grid = (N // _BLK,)
  return pl.pallas_call(
      _tc_dense_body,
      grid=grid,
      in_specs=[
          pl.BlockSpec((NC, _BLK, D), lambda ib: (0, ib, 0)),
          pl.BlockSpec((_BLK, D), lambda ib: (ib, 0)),
          pl.BlockSpec((D, D), lambda ib: (0, 0)),
          pl.BlockSpec((D,), lambda ib: (0,)),
          pl.BlockSpec((D, D), lambda ib: (0, 0)),
          pl.BlockSpec((D,), lambda ib: (0,)),
      ],
      out_specs=[
          pl.BlockSpec((_BLK, D), lambda ib: (ib, 0)),
          pl.BlockSpec((_BLK, D), lambda ib: (ib, 0)),
      ],
      out_shape=[
          jax.ShapeDtypeStruct((N, D), jnp.float32),
          jax.ShapeDtypeStruct((N, D), jnp.float32),
      ],
  )(parts, emb, w1t, b1, w2t, b2)


@functools.partial(
    pl.kernel,
    out_type=(jax.ShapeDtypeStruct((B,), jnp.float32),
              jax.ShapeDtypeStruct((B,), jnp.float32)),
    mesh=_mesh,
    compiler_params=pltpu.CompilerParams(needs_layout_passes=False),
    scratch_types=[
        pltpu.VMEM((BPT,), jnp.int32),
        pltpu.VMEM((BPT,), jnp.int32),
        pltpu.VMEM((BPT,), jnp.int32),
        pltpu.VMEM((BPT,), jnp.int32),
        [pltpu.VMEM((BCH, D), jnp.float32) for _ in range(8)],
        pltpu.VMEM((BCH, 17), jnp.float32),
        pltpu.VMEM((BCH, 17), jnp.float32),
        pltpu.VMEM((BCH,), jnp.float32),
        pltpu.VMEM((BCH,), jnp.float32),
        [pltpu.SemaphoreType.DMA for _ in range(8)],
    ],
)
def _sc_dots(e0_hbm, e1_hbm, e2_hbm, u_hbm, i_hbm, j_hbm, c_hbm,
             oi_hbm, oj_hbm,
             uv, iv, jv, cv, bufs, ai, aj, oi, oj, sems):
  cid = lax.axis_index("c")
  sid = lax.axis_index("s")
  wid = sid * NC + cid
  bbase = wid * BPT

  pltpu.sync_copy(u_hbm.at[pl.ds(bbase, BPT)], uv)
  pltpu.sync_copy(i_hbm.at[pl.ds(bbase, BPT)], iv)
  pltpu.sync_copy(j_hbm.at[pl.ds(bbase, BPT)], jv)
  pltpu.sync_copy(c_hbm.at[pl.ds(bbase, BPT)], cv)

  idxs = (uv, iv, jv, cv)
  tbls = (e0_hbm, e1_hbm, e2_hbm)
  steps = [(ch, p) for ch in range(NBCH) for p in range(3)]
  descs = {}

  def _issue(t):
    ch, p = steps[t]
    par = t % 2
    for s in range(4):
      d = pltpu.async_copy(
          tbls[p].at[idxs[s].at[pl.ds(ch * BCH, BCH)]],
          bufs[par * 4 + s], sems[par * 4 + s])
      descs[(t, s)] = d

  lanes = lax.broadcasted_iota(jnp.int32, (16,), 0)
  _issue(0)
  for t in range(len(steps)):
    ch, p = steps[t]
    par = t % 2
    if t + 1 < len(steps):
      _issue(t + 1)
    for s in range(4):
      descs.pop((t, s)).wait()
    ru, ri, rj, rc = bufs[par * 4: par * 4 + 4]

    @plsc.parallel_loop(0, BCH, unroll=2)
    def _dot(b, ru=ru, ri=ri, rj=rj, rc=rc, first=(p == 0)):
      di = _zeros16()
      dj = _zeros16()
      for k in range(D // 16):
        sl = pl.ds(k * 16, 16)
        xu = ru[b, sl]
        xi = ri[b, sl]
        xj = rj[b, sl]
        xc = rc[b, sl]
        xuc = xu + xc
        di = di + (xi * xuc + xu * xc)
        dj = dj + (xj * xuc + xu * xc)
      if first:
        ai[b, pl.ds(0, 16)] = di
        aj[b, pl.ds(0, 16)] = dj
      else:
        plsc.addupdate(ai.at[b, pl.ds(0, 16)], di)
        plsc.addupdate(aj.at[b, pl.ds(0, 16)], dj)

    if p == 2:
      for g in range(BCH // 16):
        b16 = g * 16 + lanes
        si = _zeros16()
        sj = _zeros16()
        for k in range(16):
          kk = lax.broadcast(jnp.int32(k), (16,))
          si = si + plsc.load_gather(ai, [b16, kk])
          sj = sj + plsc.load_gather(aj, [b16, kk])
        oi[pl.ds(g * 16, 16)] = si
        oj[pl.ds(g * 16, 16)] = sj
      off = bbase + ch * BCH
      pltpu.sync_copy(oi, oi_hbm.at[pl.ds(off, BCH)])
      pltpu.sync_copy(oj, oj_hbm.at[pl.ds(off, BCH)])


def kernel(u, i, j, c, row, col, val, embeddings, W1_w, W1_b, W2_w, W2_b):
  e0 = embeddings
  p1 = _sc_scatter(e0, col, row, val)
  emb1, e1n = _tc_dense(p1, e0, W1_w[0].T, W1_b[0], W2_w[0].T, W2_b[0])
  p2 = _sc_scatter(emb1, col, row, val)
  _, e2n = _tc_dense(p2, emb1, W1_w[1].T, W1_b[1], W2_w[1].T, W2_b[1])
  pred_i, pred_j = _sc_dots(e0, e1n, e2n, u, i, j, c)
  return (pred_i, pred_j)


# speedup vs baseline: 10.4336x; 1.0167x over previous
import functools

import jax
import jax.numpy as jnp
from jax import lax
from jax.experimental import pallas as pl
from jax.experimental.pallas import tpu as pltpu
from jax.experimental.pallas import tpu_sc as plsc

N = 10000
E = 320000
D = 128
B = 16384

NC = 2
NS = 16
NW = NC * NS

EPT = E // NW
ECH = 128
NFULL = EPT // ECH
ETAIL = EPT - NFULL * ECH
ROWS_PER_SUB = 624
ROWS_TAIL = N - NS * ROWS_PER_SUB

BPT = B // NW
BCH = 64
NBCH = BPT // BCH

_mesh = plsc.VectorSubcoreMesh(core_axis_name="c", subcore_axis_name="s")


def _zeros16():
  return jnp.zeros((16,), jnp.float32)


@functools.partial(
    pl.kernel,
    out_type=jax.ShapeDtypeStruct((NC, N, D), jnp.float32),
    mesh=_mesh,
    compiler_params=pltpu.CompilerParams(needs_layout_passes=False),
    scratch_types=[
        [pltpu.VMEM((ECH, D), jnp.float32) for _ in range(3)],
        [pltpu.VMEM((ECH,), jnp.int32) for _ in range(3)],
        [pltpu.VMEM((ECH,), jnp.int32) for _ in range(3)],
        [pltpu.VMEM((ECH,), jnp.float32) for _ in range(3)],
        pltpu.VMEM((ETAIL,), jnp.int32),
        pltpu.VMEM((ETAIL,), jnp.int32),
        pltpu.VMEM((ETAIL,), jnp.float32),
        pltpu.VMEM_SHARED((N, D), jnp.float32),
        [pltpu.SemaphoreType.DMA for _ in range(3)],
        [pltpu.SemaphoreType.DMA for _ in range(3)],
        [pltpu.SemaphoreType.DMA for _ in range(3)],
        [pltpu.SemaphoreType.DMA for _ in range(3)],
        pltpu.SemaphoreType.DMA,
        pltpu.SemaphoreType.DMA,
        pltpu.SemaphoreType.DMA,
        pltpu.SemaphoreType.DMA,
    ],
)
def _sc_scatter(emb_hbm, col_hbm, row_hbm, val_hbm, out_hbm,
                rows, cidx, ridx, valc, cidxt, ridxt, valct, acc,
                gsem, ssem, csem, rsem, gsemt, ssemt, csemt, rsemt):
  cid = lax.axis_index("c")
  sid = lax.axis_index("s")
  wid = sid * NC + cid
  rows0 = rows[0]

  def _zbody(e, _):
    for k in range(D // 16):
      rows0[e, pl.ds(k * 16, 16)] = _zeros16()
    return _
  lax.fori_loop(0, ECH, _zbody, 0)

  arow = sid * ROWS_PER_SUB
  for m in range(ROWS_PER_SUB // ECH):
    pltpu.sync_copy(rows0, acc.at[pl.ds(arow + m * ECH, ECH)])
  rem = ROWS_PER_SUB - (ROWS_PER_SUB // ECH) * ECH
  if rem:
    pltpu.sync_copy(rows0.at[pl.ds(0, rem)],
                    acc.at[pl.ds(arow + (ROWS_PER_SUB // ECH) * ECH, rem)])

  @pl.when(sid == NS - 1)
  def _ztail():
    pltpu.sync_copy(rows0.at[pl.ds(0, ROWS_TAIL)],
                    acc.at[pl.ds(NS * ROWS_PER_SUB, ROWS_TAIL)])

  ebase = wid * EPT
  plsc.subcore_barrier()

  def _start_cv(ch, b):
    off = ebase + ch * ECH
    pltpu.async_copy(col_hbm.at[pl.ds(off, ECH)], cidx[b], csem[b])
    pltpu.async_copy(val_hbm.at[pl.ds(off, ECH)], valc[b], csem[b])

  def _wait_cv(ch, b):
    off = ebase + ch * ECH
    pltpu.make_async_copy(col_hbm.at[pl.ds(off, ECH)], cidx[b],
                          csem[b]).wait()
    pltpu.make_async_copy(val_hbm.at[pl.ds(off, ECH)], valc[b],
                          csem[b]).wait()

  def _start_ridx(ch, b):
    pltpu.async_copy(row_hbm.at[pl.ds(ebase + ch * ECH, ECH)], ridx[b],
                     rsem[b])

  def _wait_ridx(ch, b):
    pltpu.make_async_copy(row_hbm.at[pl.ds(ebase + ch * ECH, ECH)], ridx[b],
                          rsem[b]).wait()

  def _start_gather(b):
    pltpu.async_copy(emb_hbm.at[cidx[b]], rows[b], gsem[b])

  def _wait_gather(b):
    pltpu.make_async_copy(emb_hbm.at[cidx[b]], rows[b], gsem[b]).wait()

  def _start_scatter(b):
    pltpu.async_copy(rows[b], acc.at[ridx[b]], ssem[b], add=True)

  def _wait_scatter(b):
    pltpu.make_async_copy(rows[b], acc.at[ridx[b]], ssem[b]).wait()

  def _scale(b):
    rb = rows[b]
    vc = valc[b]

    @plsc.parallel_loop(0, ECH, unroll=4)
    def _body(e):
      vb = plsc.load_gather(vc, [lax.broadcast(e, (16,))])
      for k in range(D // 16):
        sl = (e, pl.ds(k * 16, 16))
        rb[sl] = rb[sl] * vb

  _start_cv(0, 0)
  _start_cv(1, 1)
  _wait_cv(0, 0)
  _start_ridx(0, 0)
  _start_gather(0)
  _wait_cv(1, 1)
  _start_ridx(1, 1)
  _start_gather(1)
  _start_cv(2, 2)
  _wait_gather(0)
  _scale(0)
  _wait_ridx(0, 0)
  _start_scatter(0)
  _wait_cv(2, 2)
  _start_ridx(2, 2)
  _start_gather(2)
  _start_cv(3, 0)
  _wait_gather(1)
  _scale(1)
  _wait_ridx(1, 1)
  _start_scatter(1)

  NITER = (NFULL - 3) // 3

  def _step(n, cur, nx1, nx2, k, tail_k):
    _wait_cv(n + 1, nx1)
    _wait_scatter(nx1)
    _start_ridx(n + 1, nx1)
    _start_gather(nx1)
    if tail_k is None:
      _start_cv(n + 2, nx2)
    else:
      @pl.when(k < tail_k)
      def _reg():
        _start_cv(n + 2, nx2)

      @pl.when(k == tail_k)
      def _tail():
        toff2 = ebase + NFULL * ECH
        pltpu.async_copy(col_hbm.at[pl.ds(toff2, ETAIL)], cidxt, csemt)
        pltpu.async_copy(val_hbm.at[pl.ds(toff2, ETAIL)], valct, csemt)
    _wait_gather(cur)
    _scale(cur)
    _wait_ridx(n, cur)
    _start_scatter(cur)

  def _iter(k, _):
    n0 = 3 * k + 2
    _step(n0, 2, 0, 1, k, None)
    _step(n0 + 1, 0, 1, 2, k, None)
    _step(n0 + 2, 1, 2, 0, k, NITER - 1)
    return _

  lax.fori_loop(0, NITER, _iter, 0)

  nl = NFULL - 1
  sl = nl % 3
  toff = ebase + NFULL * ECH
  pltpu.make_async_copy(col_hbm.at[pl.ds(toff, ETAIL)], cidxt, csemt).wait()
  pltpu.make_async_copy(val_hbm.at[pl.ds(toff, ETAIL)], valct, csemt).wait()
  _wait_scatter((nl + 1) % 3)
  rowst = rows[(nl + 1) % 3].at[pl.ds(0, ETAIL)]
  pltpu.async_copy(row_hbm.at[pl.ds(toff, ETAIL)], ridxt, rsemt)
  pltpu.async_copy(emb_hbm.at[cidxt], rowst, gsemt)
  _wait_gather(sl)
  _scale(sl)
  _wait_ridx(nl, sl)
  _start_scatter(sl)
  pltpu.make_async_copy(emb_hbm.at[cidxt], rowst, gsemt).wait()
  rbt = rows[(nl + 1) % 3]

  @plsc.parallel_loop(0, ETAIL, unroll=2)
  def _tscale(e):
    vb = plsc.load_gather(valct, [lax.broadcast(e, (16,))])
    for k in range(D // 16):
      sl2 = (e, pl.ds(k * 16, 16))
      rbt[sl2] = rbt[sl2] * vb

  pltpu.make_async_copy(row_hbm.at[pl.ds(toff, ETAIL)], ridxt, rsemt).wait()
  pltpu.async_copy(rowst, acc.at[ridxt], ssemt, add=True).wait()
  _wait_scatter((nl + 2) % 3)
  _wait_scatter(sl)

  plsc.subcore_barrier()

  pltpu.sync_copy(acc.at[pl.ds(arow, ROWS_PER_SUB)],
                  out_hbm.at[cid, pl.ds(arow, ROWS_PER_SUB)])

  @pl.when(sid == NS - 1)
  def _otail():
    pltpu.sync_copy(acc.at[pl.ds(NS * ROWS_PER_SUB, ROWS_TAIL)],
                    out_hbm.at[cid, pl.ds(NS * ROWS_PER_SUB, ROWS_TAIL)])


_BLK = 1000


def _tc_dense_body(p_ref, emb_ref, w1_ref, b1_ref, w2_ref, b2_ref,
                   act_ref, nrm_ref):
  t1e = p_ref[0] + p_ref[1]
  emb = emb_ref[...]
  t1 = jnp.dot(t1e, w1_ref[...], preferred_element_type=jnp.float32)
  t2 = jnp.dot(emb * t1e, w2_ref[...], preferred_element_type=jnp.float32)
  a = t1 + t2 + (b1_ref[...] + b2_ref[...])[None, :]
  act = jnp.where(a >= 0, a, 0.01 * a)
  act_ref[...] = act
  nr = jnp.sqrt(jnp.sum(act * act, axis=1, keepdims=True))
  nrm_ref[...] = act / jnp.maximum(nr, 1e-12)


def _tc_dense(parts, emb, w1t, b1, w2t, b2):
  grid = (N // _BLK,)
  return pl.pallas_call(
      _tc_dense_body,
      grid=grid,
      in_specs=[
          pl.BlockSpec((NC, _BLK, D), lambda ib: (0, ib, 0)),
          pl.BlockSpec((_BLK, D), lambda ib: (ib, 0)),
          pl.BlockSpec((D, D), lambda ib: (0, 0)),
          pl.BlockSpec((D,), lambda ib: (0,)),
          pl.BlockSpec((D, D), lambda ib: (0, 0)),
          pl.BlockSpec((D,), lambda ib: (0,)),
      ],
      out_specs=[
          pl.BlockSpec((_BLK, D), lambda ib: (ib, 0)),
          pl.BlockSpec((_BLK, D), lambda ib: (ib, 0)),
      ],
      out_shape=[
          jax.ShapeDtypeStruct((N, D), jnp.float32),
          jax.ShapeDtypeStruct((N, D), jnp.float32),
      ],
  )(parts, emb, w1t, b1, w2t, b2)


@functools.partial(
    pl.kernel,
    out_type=(jax.ShapeDtypeStruct((B,), jnp.float32),
              jax.ShapeDtypeStruct((B,), jnp.float32)),
    mesh=_mesh,
    compiler_params=pltpu.CompilerParams(needs_layout_passes=False),
    scratch_types=[
        pltpu.VMEM((BPT,), jnp.int32),
        pltpu.VMEM((BPT,), jnp.int32),
        pltpu.VMEM((BPT,), jnp.int32),
        pltpu.VMEM((BPT,), jnp.int32),
        [pltpu.VMEM((BCH, D), jnp.float32) for _ in range(8)],
        pltpu.VMEM((BCH, 17), jnp.float32),
        pltpu.VMEM((BCH, 17), jnp.float32),
        [pltpu.VMEM((BCH,), jnp.float32) for _ in range(2)],
        [pltpu.VMEM((BCH,), jnp.float32) for _ in range(2)],
        [pltpu.SemaphoreType.DMA for _ in range(8)],
        [pltpu.SemaphoreType.DMA for _ in range(2)],
        [pltpu.SemaphoreType.DMA for _ in range(2)],
    ],
)
def _sc_dots(e0_hbm, e1_hbm, e2_hbm, u_hbm, i_hbm, j_hbm, c_hbm,
             oi_hbm, oj_hbm,
             uv, iv, jv, cv, bufs, ai, aj, oi, oj, sems, osi, osj):
  cid = lax.axis_index("c")
  sid = lax.axis_index("s")
  wid = sid * NC + cid
  bbase = wid * BPT

  pltpu.async_copy(u_hbm.at[pl.ds(bbase, BPT)], uv, sems[0])
  pltpu.async_copy(i_hbm.at[pl.ds(bbase, BPT)], iv, sems[1])
  pltpu.async_copy(j_hbm.at[pl.ds(bbase, BPT)], jv, sems[2])
  pltpu.async_copy(c_hbm.at[pl.ds(bbase, BPT)], cv, sems[3])
  pltpu.make_async_copy(u_hbm.at[pl.ds(bbase, BPT)], uv, sems[0]).wait()
  pltpu.make_async_copy(i_hbm.at[pl.ds(bbase, BPT)], iv, sems[1]).wait()
  pltpu.make_async_copy(j_hbm.at[pl.ds(bbase, BPT)], jv, sems[2]).wait()
  pltpu.make_async_copy(c_hbm.at[pl.ds(bbase, BPT)], cv, sems[3]).wait()

  idxs = (uv, iv, jv, cv)
  tbls = (e0_hbm, e1_hbm, e2_hbm)
  steps = [(ch, p) for ch in range(NBCH) for p in range(3)]
  descs = {}

  def _issue(t):
    ch, p = steps[t]
    par = t % 2
    for s in range(4):
      d = pltpu.async_copy(
          tbls[p].at[idxs[s].at[pl.ds(ch * BCH, BCH)]],
          bufs[par * 4 + s], sems[par * 4 + s])
      descs[(t, s)] = d

  lanes = lax.broadcasted_iota(jnp.int32, (16,), 0)
  _issue(0)
  for t in range(len(steps)):
    ch, p = steps[t]
    par = t % 2
    if t + 1 < len(steps):
      _issue(t + 1)
    for s in range(4):
      descs.pop((t, s)).wait()
    ru, ri, rj, rc = bufs[par * 4: par * 4 + 4]

    @plsc.parallel_loop(0, BCH, unroll=2)
    def _dot(b, ru=ru, ri=ri, rj=rj, rc=rc, first=(p == 0)):
      di = _zeros16()
      dj = _zeros16()
      for k in range(D // 16):
        sl = pl.ds(k * 16, 16)
        xu = ru[b, sl]
        xi = ri[b, sl]
        xj = rj[b, sl]
        xc = rc[b, sl]
        xuc = xu + xc
        di = di + (xi * xuc + xu * xc)
        dj = dj + (xj * xuc + xu * xc)
      if first:
        ai[b, pl.ds(0, 16)] = di
        aj[b, pl.ds(0, 16)] = dj
      else:
        plsc.addupdate(ai.at[b, pl.ds(0, 16)], di)
        plsc.addupdate(aj.at[b, pl.ds(0, 16)], dj)

    if p == 2:
      ob = ch % 2
      if ch >= 2:
        off2 = bbase + (ch - 2) * BCH
        pltpu.make_async_copy(oi[ob], oi_hbm.at[pl.ds(off2, BCH)],
                              osi[ob]).wait()
        pltpu.make_async_copy(oj[ob], oj_hbm.at[pl.ds(off2, BCH)],
                              osj[ob]).wait()
      for g in range(BCH // 16):
        b16 = g * 16 + lanes
        si = _zeros16()
        sj = _zeros16()
        for k in range(16):
          kk = lax.broadcast(jnp.int32(k), (16,))
          si = si + plsc.load_gather(ai, [b16, kk])
          sj = sj + plsc.load_gather(aj, [b16, kk])
        oi[ob][pl.ds(g * 16, 16)] = si
        oj[ob][pl.ds(g * 16, 16)] = sj
      off = bbase + ch * BCH
      pltpu.async_copy(oi[ob], oi_hbm.at[pl.ds(off, BCH)], osi[ob])
      pltpu.async_copy(oj[ob], oj_hbm.at[pl.ds(off, BCH)], osj[ob])

  for ch in (NBCH - 2, NBCH - 1):
    ob = ch % 2
    off2 = bbase + ch * BCH
    pltpu.make_async_copy(oi[ob], oi_hbm.at[pl.ds(off2, BCH)], osi[ob]).wait()
    pltpu.make_async_copy(oj[ob], oj_hbm.at[pl.ds(off2, BCH)], osj[ob]).wait()


def kernel(u, i, j, c, row, col, val, embeddings, W1_w, W1_b, W2_w, W2_b):
  e0 = embeddings
  p1 = _sc_scatter(e0, col, row, val)
  emb1, e1n = _tc_dense(p1, e0, W1_w[0].T, W1_b[0], W2_w[0].T, W2_b[0])
  p2 = _sc_scatter(emb1, col, row, val)
  _, e2n = _tc_dense(p2, emb1, W1_w[1].T, W1_b[1], W2_w[1].T, W2_b[1])
  pred_i, pred_j = _sc_dots(e0, e1n, e2n, u, i, j, c)
  return (pred_i, pred_j)


# speedup vs baseline: 10.5506x; 1.0112x over previous
import functools

import jax
import jax.numpy as jnp
from jax import lax
from jax.experimental import pallas as pl
from jax.experimental.pallas import tpu as pltpu
from jax.experimental.pallas import tpu_sc as plsc

N = 10000
E = 320000
D = 128
B = 16384

NC = 2
NS = 16
NW = NC * NS

EPT = E // NW
ECH = 128
NFULL = EPT // ECH
ETAIL = EPT - NFULL * ECH
ROWS_PER_SUB = 624
ROWS_TAIL = N - NS * ROWS_PER_SUB

BPT = B // NW
BCH = 64
NBCH = BPT // BCH

_mesh = plsc.VectorSubcoreMesh(core_axis_name="c", subcore_axis_name="s")


def _zeros16():
  return jnp.zeros((16,), jnp.float32)


@functools.partial(
    pl.kernel,
    out_type=jax.ShapeDtypeStruct((NC, N, D), jnp.float32),
    mesh=_mesh,
    compiler_params=pltpu.CompilerParams(needs_layout_passes=False),
    scratch_types=[
        [pltpu.VMEM((ECH, D), jnp.float32) for _ in range(3)],
        [pltpu.VMEM((ECH,), jnp.int32) for _ in range(3)],
        [pltpu.VMEM((ECH,), jnp.int32) for _ in range(3)],
        [pltpu.VMEM((ECH,), jnp.float32) for _ in range(3)],
        pltpu.VMEM((ETAIL,), jnp.int32),
        pltpu.VMEM((ETAIL,), jnp.int32),
        pltpu.VMEM((ETAIL,), jnp.float32),
        pltpu.VMEM_SHARED((N, D), jnp.float32),
        [pltpu.SemaphoreType.DMA for _ in range(3)],
        [pltpu.SemaphoreType.DMA for _ in range(3)],
        [pltpu.SemaphoreType.DMA for _ in range(3)],
        [pltpu.SemaphoreType.DMA for _ in range(3)],
        pltpu.SemaphoreType.DMA,
        pltpu.SemaphoreType.DMA,
        pltpu.SemaphoreType.DMA,
        pltpu.SemaphoreType.DMA,
        pltpu.SemaphoreType.DMA,
    ],
)
def _sc_scatter(emb_hbm, col_hbm, row_hbm, val_hbm, out_hbm,
                rows, cidx, ridx, valc, cidxt, ridxt, valct, acc,
                gsem, ssem, csem, rsem, gsemt, ssemt, csemt, rsemt, zsem):
  cid = lax.axis_index("c")
  sid = lax.axis_index("s")
  wid = sid * NC + cid
  ebase = wid * EPT
  arow = sid * ROWS_PER_SUB

  def _start_cv(ch, b):
    off = ebase + ch * ECH
    pltpu.async_copy(col_hbm.at[pl.ds(off, ECH)], cidx[b], csem[b])
    pltpu.async_copy(val_hbm.at[pl.ds(off, ECH)], valc[b], csem[b])

  def _wait_cv(ch, b):
    off = ebase + ch * ECH
    pltpu.make_async_copy(col_hbm.at[pl.ds(off, ECH)], cidx[b],
                          csem[b]).wait()
    pltpu.make_async_copy(val_hbm.at[pl.ds(off, ECH)], valc[b],
                          csem[b]).wait()

  def _start_ridx(ch, b):
    pltpu.async_copy(row_hbm.at[pl.ds(ebase + ch * ECH, ECH)], ridx[b],
                     rsem[b])

  def _wait_ridx(ch, b):
    pltpu.make_async_copy(row_hbm.at[pl.ds(ebase + ch * ECH, ECH)], ridx[b],
                          rsem[b]).wait()

  def _start_gather(b):
    pltpu.async_copy(emb_hbm.at[cidx[b]], rows[b], gsem[b])

  def _wait_gather(b):
    pltpu.make_async_copy(emb_hbm.at[cidx[b]], rows[b], gsem[b]).wait()

  def _start_scatter(b):
    pltpu.async_copy(rows[b], acc.at[ridx[b]], ssem[b], add=True)

  def _wait_scatter(b):
    pltpu.make_async_copy(rows[b], acc.at[ridx[b]], ssem[b]).wait()

  def _scale(b):
    rb = rows[b]
    vc = valc[b]

    @plsc.parallel_loop(0, ECH, unroll=4)
    def _body(e):
      vb = plsc.load_gather(vc, [lax.broadcast(e, (16,))])
      for k in range(D // 16):
        sl = (e, pl.ds(k * 16, 16))
        rb[sl] = rb[sl] * vb

  _start_cv(0, 0)
  _start_cv(1, 1)

  rows2 = rows[2]

  def _zbody(e, _):
    for k in range(D // 16):
      rows2[e, pl.ds(k * 16, 16)] = _zeros16()
    return _
  lax.fori_loop(0, ECH, _zbody, 0)

  NZ = ROWS_PER_SUB // ECH
  ZREM = ROWS_PER_SUB - NZ * ECH
  for m in range(NZ):
    pltpu.async_copy(rows2, acc.at[pl.ds(arow + m * ECH, ECH)], zsem)
  if ZREM:
    pltpu.async_copy(rows2.at[pl.ds(0, ZREM)],
                     acc.at[pl.ds(arow + NZ * ECH, ZREM)], zsem)

  @pl.when(sid == NS - 1)
  def _ztail():
    pltpu.async_copy(rows2.at[pl.ds(0, ROWS_TAIL)],
                     acc.at[pl.ds(NS * ROWS_PER_SUB, ROWS_TAIL)], zsem)

  _wait_cv(0, 0)
  _start_ridx(0, 0)
  _start_gather(0)
  _wait_cv(1, 1)
  _start_ridx(1, 1)
  _start_gather(1)
  _start_cv(2, 2)

  for m in range(NZ):
    pltpu.make_async_copy(rows2, acc.at[pl.ds(arow + m * ECH, ECH)],
                          zsem).wait()
  if ZREM:
    pltpu.make_async_copy(rows2.at[pl.ds(0, ZREM)],
                          acc.at[pl.ds(arow + NZ * ECH, ZREM)], zsem).wait()

  @pl.when(sid == NS - 1)
  def _ztailw():
    pltpu.make_async_copy(rows2.at[pl.ds(0, ROWS_TAIL)],
                          acc.at[pl.ds(NS * ROWS_PER_SUB, ROWS_TAIL)],
                          zsem).wait()

  plsc.subcore_barrier()

  _wait_gather(0)
  _scale(0)
  _wait_ridx(0, 0)
  _start_scatter(0)
  _wait_cv(2, 2)
  _start_ridx(2, 2)
  _start_gather(2)
  _start_cv(3, 0)
  _wait_gather(1)
  _scale(1)
  _wait_ridx(1, 1)
  _start_scatter(1)

  NITER = (NFULL - 3) // 3

  def _step(n, cur, nx1, nx2, k, tail_k):
    _wait_cv(n + 1, nx1)
    _wait_scatter(nx1)
    _start_ridx(n + 1, nx1)
    _start_gather(nx1)
    if tail_k is None:
      _start_cv(n + 2, nx2)
    else:
      @pl.when(k < tail_k)
      def _reg():
        _start_cv(n + 2, nx2)

      @pl.when(k == tail_k)
      def _tail():
        toff2 = ebase + NFULL * ECH
        pltpu.async_copy(col_hbm.at[pl.ds(toff2, ETAIL)], cidxt, csemt)
        pltpu.async_copy(val_hbm.at[pl.ds(toff2, ETAIL)], valct, csemt)
    _wait_gather(cur)
    _scale(cur)
    _wait_ridx(n, cur)
    _start_scatter(cur)

  def _iter(k, _):
    n0 = 3 * k + 2
    _step(n0, 2, 0, 1, k, None)
    _step(n0 + 1, 0, 1, 2, k, None)
    _step(n0 + 2, 1, 2, 0, k, NITER - 1)
    return _

  lax.fori_loop(0, NITER, _iter, 0)

  nl = NFULL - 1
  sl = nl % 3
  toff = ebase + NFULL * ECH
  pltpu.make_async_copy(col_hbm.at[pl.ds(toff, ETAIL)], cidxt, csemt).wait()
  pltpu.make_async_copy(val_hbm.at[pl.ds(toff, ETAIL)], valct, csemt).wait()
  _wait_scatter((nl + 1) % 3)
  rowst = rows[(nl + 1) % 3].at[pl.ds(0, ETAIL)]
  pltpu.async_copy(row_hbm.at[pl.ds(toff, ETAIL)], ridxt, rsemt)
  pltpu.async_copy(emb_hbm.at[cidxt], rowst, gsemt)
  _wait_gather(sl)
  _scale(sl)
  _wait_ridx(nl, sl)
  _start_scatter(sl)
  pltpu.make_async_copy(emb_hbm.at[cidxt], rowst, gsemt).wait()
  rbt = rows[(nl + 1) % 3]

  @plsc.parallel_loop(0, ETAIL, unroll=2)
  def _tscale(e):
    vb = plsc.load_gather(valct, [lax.broadcast(e, (16,))])
    for k in range(D // 16):
      sl2 = (e, pl.ds(k * 16, 16))
      rbt[sl2] = rbt[sl2] * vb

  pltpu.make_async_copy(row_hbm.at[pl.ds(toff, ETAIL)], ridxt, rsemt).wait()
  pltpu.async_copy(rowst, acc.at[ridxt], ssemt, add=True).wait()
  _wait_scatter((nl + 2) % 3)
  _wait_scatter(sl)

  plsc.subcore_barrier()

  pltpu.sync_copy(acc.at[pl.ds(arow, ROWS_PER_SUB)],
                  out_hbm.at[cid, pl.ds(arow, ROWS_PER_SUB)])

  @pl.when(sid == NS - 1)
  def _otail():
    pltpu.sync_copy(acc.at[pl.ds(NS * ROWS_PER_SUB, ROWS_TAIL)],
                    out_hbm.at[cid, pl.ds(NS * ROWS_PER_SUB, ROWS_TAIL)])


_BLK = 1000


def _tc_dense_body(p_ref, emb_ref, w1_ref, b1_ref, w2_ref, b2_ref,
                   act_ref, nrm_ref):
  t1e = p_ref[0] + p_ref[1]
  emb = emb_ref[...]
  t1 = jnp.dot(t1e, w1_ref[...], preferred_element_type=jnp.float32)
  t2 = jnp.dot(emb * t1e, w2_ref[...], preferred_element_type=jnp.float32)
  a = t1 + t2 + (b1_ref[...] + b2_ref[...])[None, :]
  act = jnp.where(a >= 0, a, 0.01 * a)
  act_ref[...] = act
  nr = jnp.sqrt(jnp.sum(act * act, axis=1, keepdims=True))
  nrm_ref[...] = act / jnp.maximum(nr, 1e-12)


def _tc_dense(parts, emb, w1t, b1, w2t, b2):
  grid = (N // _BLK,)
  return pl.pallas_call(
      _tc_dense_body,
      grid=grid,
      in_specs=[
          pl.BlockSpec((NC, _BLK, D), lambda ib: (0, ib, 0)),
          pl.BlockSpec((_BLK, D), lambda ib: (ib, 0)),
          pl.BlockSpec((D, D), lambda ib: (0, 0)),
          pl.BlockSpec((D,), lambda ib: (0,)),
          pl.BlockSpec((D, D), lambda ib: (0, 0)),
          pl.BlockSpec((D,), lambda ib: (0,)),
      ],
      out_specs=[
          pl.BlockSpec((_BLK, D), lambda ib: (ib, 0)),
          pl.BlockSpec((_BLK, D), lambda ib: (ib, 0)),
      ],
      out_shape=[
          jax.ShapeDtypeStruct((N, D), jnp.float32),
          jax.ShapeDtypeStruct((N, D), jnp.float32),
      ],
  )(parts, emb, w1t, b1, w2t, b2)


@functools.partial(
    pl.kernel,
    out_type=(jax.ShapeDtypeStruct((B,), jnp.float32),
              jax.ShapeDtypeStruct((B,), jnp.float32)),
    mesh=_mesh,
    compiler_params=pltpu.CompilerParams(needs_layout_passes=False),
    scratch_types=[
        pltpu.VMEM((BPT,), jnp.int32),
        pltpu.VMEM((BPT,), jnp.int32),
        pltpu.VMEM((BPT,), jnp.int32),
        pltpu.VMEM((BPT,), jnp.int32),
        [pltpu.VMEM((BCH, D), jnp.float32) for _ in range(8)],
        pltpu.VMEM((BCH, 17), jnp.float32),
        pltpu.VMEM((BCH, 17), jnp.float32),
        [pltpu.VMEM((BCH,), jnp.float32) for _ in range(2)],
        [pltpu.VMEM((BCH,), jnp.float32) for _ in range(2)],
        [pltpu.SemaphoreType.DMA for _ in range(8)],
        [pltpu.SemaphoreType.DMA for _ in range(2)],
        [pltpu.SemaphoreType.DMA for _ in range(2)],
    ],
)
def _sc_dots(e0_hbm, e1_hbm, e2_hbm, u_hbm, i_hbm, j_hbm, c_hbm,
             oi_hbm, oj_hbm,
             uv, iv, jv, cv, bufs, ai, aj, oi, oj, sems, osi, osj):
  cid = lax.axis_index("c")
  sid = lax.axis_index("s")
  wid = sid * NC + cid
  bbase = wid * BPT

  pltpu.async_copy(u_hbm.at[pl.ds(bbase, BPT)], uv, sems[0])
  pltpu.async_copy(i_hbm.at[pl.ds(bbase, BPT)], iv, sems[1])
  pltpu.async_copy(j_hbm.at[pl.ds(bbase, BPT)], jv, sems[2])
  pltpu.async_copy(c_hbm.at[pl.ds(bbase, BPT)], cv, sems[3])
  pltpu.make_async_copy(u_hbm.at[pl.ds(bbase, BPT)], uv, sems[0]).wait()
  pltpu.make_async_copy(i_hbm.at[pl.ds(bbase, BPT)], iv, sems[1]).wait()
  pltpu.make_async_copy(j_hbm.at[pl.ds(bbase, BPT)], jv, sems[2]).wait()
  pltpu.make_async_copy(c_hbm.at[pl.ds(bbase, BPT)], cv, sems[3]).wait()

  idxs = (uv, iv, jv, cv)
  tbls = (e0_hbm, e1_hbm, e2_hbm)
  steps = [(ch, p) for ch in range(NBCH) for p in range(3)]
  descs = {}

  def _issue(t):
    ch, p = steps[t]
    par = t % 2
    for s in range(4):
      d = pltpu.async_copy(
          tbls[p].at[idxs[s].at[pl.ds(ch * BCH, BCH)]],
          bufs[par * 4 + s], sems[par * 4 + s])
      descs[(t, s)] = d

  lanes = lax.broadcasted_iota(jnp.int32, (16,), 0)
  _issue(0)
  for t in range(len(steps)):
    ch, p = steps[t]
    par = t % 2
    if t + 1 < len(steps):
      _issue(t + 1)
    for s in range(4):
      descs.pop((t, s)).wait()
    ru, ri, rj, rc = bufs[par * 4: par * 4 + 4]

    @plsc.parallel_loop(0, BCH, unroll=2)
    def _dot(b, ru=ru, ri=ri, rj=rj, rc=rc, first=(p == 0)):
      di = _zeros16()
      dj = _zeros16()
      for k in range(D // 16):
        sl = pl.ds(k * 16, 16)
        xu = ru[b, sl]
        xi = ri[b, sl]
        xj = rj[b, sl]
        xc = rc[b, sl]
        xuc = xu + xc
        di = di + (xi * xuc + xu * xc)
        dj = dj + (xj * xuc + xu * xc)
      if first:
        ai[b, pl.ds(0, 16)] = di
        aj[b, pl.ds(0, 16)] = dj
      else:
        plsc.addupdate(ai.at[b, pl.ds(0, 16)], di)
        plsc.addupdate(aj.at[b, pl.ds(0, 16)], dj)

    if p == 2:
      ob = ch % 2
      if ch >= 2:
        off2 = bbase + (ch - 2) * BCH
        pltpu.make_async_copy(oi[ob], oi_hbm.at[pl.ds(off2, BCH)],
                              osi[ob]).wait()
        pltpu.make_async_copy(oj[ob], oj_hbm.at[pl.ds(off2, BCH)],
                              osj[ob]).wait()
      for g in range(BCH // 16):
        b16 = g * 16 + lanes
        si = _zeros16()
        sj = _zeros16()
        for k in range(16):
          kk = lax.broadcast(jnp.int32(k), (16,))
          si = si + plsc.load_gather(ai, [b16, kk])
          sj = sj + plsc.load_gather(aj, [b16, kk])
        oi[ob][pl.ds(g * 16, 16)] = si
        oj[ob][pl.ds(g * 16, 16)] = sj
      off = bbase + ch * BCH
      pltpu.async_copy(oi[ob], oi_hbm.at[pl.ds(off, BCH)], osi[ob])
      pltpu.async_copy(oj[ob], oj_hbm.at[pl.ds(off, BCH)], osj[ob])

  for ch in (NBCH - 2, NBCH - 1):
    ob = ch % 2
    off2 = bbase + ch * BCH
    pltpu.make_async_copy(oi[ob], oi_hbm.at[pl.ds(off2, BCH)], osi[ob]).wait()
    pltpu.make_async_copy(oj[ob], oj_hbm.at[pl.ds(off2, BCH)], osj[ob]).wait()


def kernel(u, i, j, c, row, col, val, embeddings, W1_w, W1_b, W2_w, W2_b):
  e0 = embeddings
  p1 = _sc_scatter(e0, col, row, val)
  emb1, e1n = _tc_dense(p1, e0, W1_w[0].T, W1_b[0], W2_w[0].T, W2_b[0])
  p2 = _sc_scatter(emb1, col, row, val)
  _, e2n = _tc_dense(p2, emb1, W1_w[1].T, W1_b[1], W2_w[1].T, W2_b[1])
  pred_i, pred_j = _sc_dots(e0, e1n, e2n, u, i, j, c)
  return (pred_i, pred_j)
